# baseline (speedup 1.0000x reference)
import functools

import jax
import jax.numpy as jnp
from jax import lax
from jax.experimental import pallas as pl
from jax.experimental.pallas import tpu as pltpu
from jax.experimental.pallas import tpu_sc as plsc


def _sc_gather(E, idx_flat):
    N = idx_flat.shape[0]
    D = E.shape[1]
    info = plsc.get_sparse_core_info()
    NW = info.num_cores * info.num_subcores
    b_per_w = N // NW
    assert N % NW == 0 and b_per_w % 8 == 0
    mesh = plsc.VectorSubcoreMesh(core_axis_name="c", subcore_axis_name="s")

    @functools.partial(
        pl.kernel,
        mesh=mesh,
        out_type=jax.ShapeDtypeStruct((N, D), jnp.float32),
        scratch_types=[
            pltpu.VMEM((b_per_w,), jnp.int32),
            pltpu.VMEM((b_per_w, D), jnp.float32),
            pltpu.SemaphoreType.DMA,
        ],
        compiler_params=pltpu.CompilerParams(use_tc_tiling_on_sc=False),
    )
    def gather_k(table_hbm, idx_hbm, out_hbm, idx_v, rows_v, sem):
        wid = lax.axis_index("s") * info.num_cores + lax.axis_index("c")
        base = wid * b_per_w
        pltpu.sync_copy(idx_hbm.at[pl.ds(base, b_per_w)], idx_v)
        pltpu.async_copy(table_hbm.at[idx_v], rows_v, sem).wait()
        pltpu.sync_copy(rows_v, out_hbm.at[pl.ds(base, b_per_w)])

    return gather_k(E, idx_flat)


def _gru_body(emb_ref, wih_ref, whh_ref, bih_ref, bhh_ref, out_ref, h_ref):
    t = pl.program_id(0)
    Hd = h_ref.shape[1]

    @pl.when(t == 0)
    def _():
        h_ref[...] = jnp.zeros_like(h_ref)

    h = h_ref[...]
    xt = emb_ref[0]
    gi = jnp.dot(xt, wih_ref[...], preferred_element_type=jnp.float32) + bih_ref[...]
    gh = jnp.dot(h, whh_ref[...], preferred_element_type=jnp.float32) + bhh_ref[...]
    i_r, i_z, i_n = gi[:, :Hd], gi[:, Hd:2 * Hd], gi[:, 2 * Hd:]
    h_r, h_z, h_n = gh[:, :Hd], gh[:, Hd:2 * Hd], gh[:, 2 * Hd:]
    r = jax.nn.sigmoid(i_r + h_r)
    z = jax.nn.sigmoid(i_z + h_z)
    n = jnp.tanh(i_n + r * h_n)
    h_new = (1.0 - z) * n + z * h
    h_ref[...] = h_new
    out_ref[0] = h_new


def _gru(emb, W_ih, W_hh, b_ih, b_hh, interpret=False):
    Lx, Bx, Hx = emb.shape
    wih_t = W_ih.T
    whh_t = W_hh.T
    bih2 = b_ih.reshape(1, 3 * Hx)
    bhh2 = b_hh.reshape(1, 3 * Hx)
    return pl.pallas_call(
        _gru_body,
        grid=(Lx,),
        in_specs=[
            pl.BlockSpec((1, Bx, Hx), lambda t: (t, 0, 0)),
            pl.BlockSpec((Hx, 3 * Hx), lambda t: (0, 0)),
            pl.BlockSpec((Hx, 3 * Hx), lambda t: (0, 0)),
            pl.BlockSpec((1, 3 * Hx), lambda t: (0, 0)),
            pl.BlockSpec((1, 3 * Hx), lambda t: (0, 0)),
        ],
        out_specs=pl.BlockSpec((1, Bx, Hx), lambda t: (t, 0, 0)),
        out_shape=jax.ShapeDtypeStruct((Lx, Bx, Hx), jnp.float32),
        scratch_shapes=[pltpu.VMEM((Bx, Hx), jnp.float32)],
        interpret=interpret,
    )(emb, wih_t, whh_t, bih2, bhh2)


def kernel(x, E, W_ih, W_hh, b_ih, b_hh):
    Lx, Bx = x.shape
    Hx = E.shape[1]
    emb = _sc_gather(E, x.reshape(-1)).reshape(Lx, Bx, Hx)
    out = _gru(emb, W_ih, W_hh, b_ih, b_hh)
    return out, out[Lx - 1:Lx]


# speedup vs baseline: 2.0816x; 2.0816x over previous
import functools

import jax
import jax.numpy as jnp
from jax import lax
from jax.experimental import pallas as pl
from jax.experimental.pallas import tpu as pltpu
from jax.experimental.pallas import tpu_sc as plsc


def _sc_gather(E, idx_flat):
    N = idx_flat.shape[0]
    D = E.shape[1]
    info = plsc.get_sparse_core_info()
    NW = info.num_cores * info.num_subcores
    b_per_w = N // NW
    assert N % NW == 0 and b_per_w % 8 == 0
    mesh = plsc.VectorSubcoreMesh(core_axis_name="c", subcore_axis_name="s")

    @functools.partial(
        pl.kernel,
        mesh=mesh,
        out_type=jax.ShapeDtypeStruct((N, D), jnp.float32),
        scratch_types=[
            pltpu.VMEM((b_per_w,), jnp.int32),
            pltpu.VMEM((b_per_w, D), jnp.float32),
            pltpu.SemaphoreType.DMA,
        ],
        compiler_params=pltpu.CompilerParams(use_tc_tiling_on_sc=False),
    )
    def gather_k(table_hbm, idx_hbm, out_hbm, idx_v, rows_v, sem):
        wid = lax.axis_index("s") * info.num_cores + lax.axis_index("c")
        base = wid * b_per_w
        pltpu.sync_copy(idx_hbm.at[pl.ds(base, b_per_w)], idx_v)
        pltpu.async_copy(table_hbm.at[idx_v], rows_v, sem).wait()
        pltpu.sync_copy(rows_v, out_hbm.at[pl.ds(base, b_per_w)])

    return gather_k(E, idx_flat)


def _gru_body(emb_ref, wih_ref, whh_ref, bih_ref, bhh_ref, out_ref, h_ref):
    t = pl.program_id(0)
    Hd = h_ref.shape[1]

    @pl.when(t == 0)
    def _():
        h_ref[...] = jnp.zeros_like(h_ref)

    h = h_ref[...]
    xt = emb_ref[0]
    gi = jnp.dot(xt, wih_ref[...], preferred_element_type=jnp.float32) + bih_ref[...]
    gh = jnp.dot(h, whh_ref[...], preferred_element_type=jnp.float32) + bhh_ref[...]
    i_r, i_z, i_n = gi[:, :Hd], gi[:, Hd:2 * Hd], gi[:, 2 * Hd:]
    h_r, h_z, h_n = gh[:, :Hd], gh[:, Hd:2 * Hd], gh[:, 2 * Hd:]
    r = jax.nn.sigmoid(i_r + h_r)
    z = jax.nn.sigmoid(i_z + h_z)
    n = jnp.tanh(i_n + r * h_n)
    h_new = (1.0 - z) * n + z * h
    h_ref[...] = h_new
    out_ref[0] = h_new


def _gru(emb, W_ih, W_hh, b_ih, b_hh, interpret=False):
    Lx, Bx, Hx = emb.shape
    wih_t = W_ih.T
    whh_t = W_hh.T
    bih2 = b_ih.reshape(1, 3 * Hx)
    bhh2 = b_hh.reshape(1, 3 * Hx)
    return pl.pallas_call(
        _gru_body,
        grid=(Lx,),
        in_specs=[
            pl.BlockSpec((1, Bx, Hx), lambda t: (t, 0, 0)),
            pl.BlockSpec((Hx, 3 * Hx), lambda t: (0, 0)),
            pl.BlockSpec((Hx, 3 * Hx), lambda t: (0, 0)),
            pl.BlockSpec((1, 3 * Hx), lambda t: (0, 0)),
            pl.BlockSpec((1, 3 * Hx), lambda t: (0, 0)),
        ],
        out_specs=pl.BlockSpec((1, Bx, Hx), lambda t: (t, 0, 0)),
        out_shape=jax.ShapeDtypeStruct((Lx, Bx, Hx), jnp.float32),
        scratch_shapes=[pltpu.VMEM((Bx, Hx), jnp.float32)],
        interpret=interpret,
    )(emb, wih_t, whh_t, bih2, bhh2)


def kernel(x, E, W_ih, W_hh, b_ih, b_hh):
    Lx, Bx = x.shape
    Hx = E.shape[1]
    emb = jnp.take(E, x.reshape(-1), axis=0).reshape(Lx, Bx, Hx)
    out = _gru(emb, W_ih, W_hh, b_ih, b_hh)
    return out, out[Lx - 1:Lx]
